# Initial kernel scaffold; baseline (speedup 1.0000x reference)
import jax
import jax.numpy as jnp
from jax.experimental import pallas as pl


def kernel(tokens, emb_table):
    raise NotImplementedError("write your pallas kernel here")


# speedup vs baseline: 3.8970x; 3.8970x over previous
import functools
import math

import jax
import jax.numpy as jnp
from jax.experimental import pallas as pl
from jax.experimental.pallas import tpu as pltpu

_VMEM_LIMIT_BYTES = 48 << 20
_TABLE_DMA_CHUNKS = 8


def _round_up(x: int, m: int) -> int:
    return (x + m - 1) // m * m


def _gather_kernel(ids_ref, emb_hbm, out_ref, tbl, sems, *,
                   tb, half, n_chunks, rows_per_chunk, scale):
    h = pl.program_id(0)
    blk = pl.program_id(1)

    @pl.when(blk == 0)
    def _load_half_table():
        col = pl.multiple_of(h * half, half)
        for c in range(n_chunks):
            pltpu.make_async_copy(
                emb_hbm.at[pl.ds(c * rows_per_chunk, rows_per_chunk), :,
                           pl.ds(col, half)],
                tbl.at[pl.ds(c * rows_per_chunk, rows_per_chunk)],
                sems.at[c],
            ).start()
        for c in range(n_chunks):
            pltpu.make_async_copy(
                emb_hbm.at[pl.ds(c * rows_per_chunk, rows_per_chunk), :,
                           pl.ds(col, half)],
                tbl.at[pl.ds(c * rows_per_chunk, rows_per_chunk)],
                sems.at[c],
            ).wait()

    base = blk * tb
    for mi in range(tb):
        tok = ids_ref[base + mi]
        out_ref[mi, 0] = tbl[tok, 0] * scale


def kernel(tokens: jax.Array, emb_table: jax.Array) -> jax.Array:
    seq_len, batch = tokens.shape
    vocab, emb = emb_table.shape
    n = seq_len * batch
    scale = math.sqrt(emb)

    emb_p = _round_up(emb, 256)
    if emb_p != emb:
        emb_table = jnp.pad(emb_table, ((0, 0), (0, emb_p - emb)))
    half = emb_p // 2

    ids = jnp.clip(tokens.reshape(n).astype(jnp.int32), 0, vocab - 1)

    tb = 512
    n_pad = _round_up(n, tb)
    if n_pad != n:
        ids = jnp.pad(ids, (0, n_pad - n))
    nb = n_pad // tb

    n_chunks = _TABLE_DMA_CHUNKS
    while vocab % n_chunks != 0:
        n_chunks //= 2
    rows_per_chunk = vocab // n_chunks

    emb3 = emb_table.reshape(vocab, 1, emb_p)

    grid_spec = pltpu.PrefetchScalarGridSpec(
        num_scalar_prefetch=1,
        grid=(2, nb),
        in_specs=[pl.BlockSpec(memory_space=pl.ANY)],
        out_specs=pl.BlockSpec((tb, 1, half), lambda h, i, ids: (i, 0, h)),
        scratch_shapes=[
            pltpu.VMEM((vocab, 1, half), emb_table.dtype),
            pltpu.SemaphoreType.DMA((n_chunks,)),
        ],
    )
    out = pl.pallas_call(
        functools.partial(_gather_kernel, tb=tb, half=half,
                          n_chunks=n_chunks, rows_per_chunk=rows_per_chunk,
                          scale=scale),
        out_shape=jax.ShapeDtypeStruct((n_pad, 1, emb_p), emb_table.dtype),
        grid_spec=grid_spec,
        compiler_params=pltpu.CompilerParams(
            dimension_semantics=("parallel", "arbitrary"),
            vmem_limit_bytes=_VMEM_LIMIT_BYTES,
        ),
    )(ids, emb3)

    return out[:n, 0, :emb].reshape(seq_len, batch, emb)


# speedup vs baseline: 4.6694x; 1.1982x over previous
import functools
import math

import jax
import jax.numpy as jnp
from jax.experimental import pallas as pl
from jax.experimental.pallas import tpu as pltpu

_VMEM_LIMIT_BYTES = 48 << 20
_TABLE_DMA_CHUNKS = 8


def _round_up(x: int, m: int) -> int:
    return (x + m - 1) // m * m


def _gather_kernel(ids_ref, emb_hbm, out_ref, tbl, gtile, sems, *,
                   tb, half, n_chunks, rows_per_chunk, scale):
    h = pl.program_id(0)
    blk = pl.program_id(1)

    @pl.when(blk == 0)
    def _load_half_table():
        col = pl.multiple_of(h * half, half)
        for c in range(n_chunks):
            pltpu.make_async_copy(
                emb_hbm.at[pl.ds(c * rows_per_chunk, rows_per_chunk), :,
                           pl.ds(col, half)],
                tbl.at[pl.ds(c * rows_per_chunk, rows_per_chunk)],
                sems.at[c],
            ).start()
        for c in range(n_chunks):
            pltpu.make_async_copy(
                emb_hbm.at[pl.ds(c * rows_per_chunk, rows_per_chunk), :,
                           pl.ds(col, half)],
                tbl.at[pl.ds(c * rows_per_chunk, rows_per_chunk)],
                sems.at[c],
            ).wait()

    base = blk * tb
    for mi in range(tb):
        tok = ids_ref[base + mi]
        gtile[mi, 0] = tbl[tok, 0]
    out_ref[...] = gtile[:, 0, :] * scale


def kernel(tokens: jax.Array, emb_table: jax.Array) -> jax.Array:
    seq_len, batch = tokens.shape
    vocab, emb = emb_table.shape
    n = seq_len * batch
    scale = math.sqrt(emb)

    emb_p = _round_up(emb, 256)
    if emb_p != emb:
        emb_table = jnp.pad(emb_table, ((0, 0), (0, emb_p - emb)))
    half = emb_p // 2

    ids = jnp.clip(tokens.reshape(n).astype(jnp.int32), 0, vocab - 1)

    tb = 512
    n_pad = _round_up(n, tb)
    if n_pad != n:
        ids = jnp.pad(ids, (0, n_pad - n))
    nb = n_pad // tb

    n_chunks = _TABLE_DMA_CHUNKS
    while vocab % n_chunks != 0:
        n_chunks //= 2
    rows_per_chunk = vocab // n_chunks

    emb3 = emb_table.reshape(vocab, 1, emb_p)

    grid_spec = pltpu.PrefetchScalarGridSpec(
        num_scalar_prefetch=1,
        grid=(2, nb),
        in_specs=[pl.BlockSpec(memory_space=pl.ANY)],
        out_specs=pl.BlockSpec((tb, half), lambda h, i, ids: (i, h)),
        scratch_shapes=[
            pltpu.VMEM((vocab, 1, half), emb_table.dtype),
            pltpu.VMEM((tb, 1, half), emb_table.dtype),
            pltpu.SemaphoreType.DMA((n_chunks,)),
        ],
    )
    out = pl.pallas_call(
        functools.partial(_gather_kernel, tb=tb, half=half,
                          n_chunks=n_chunks, rows_per_chunk=rows_per_chunk,
                          scale=scale),
        out_shape=jax.ShapeDtypeStruct((n_pad, emb_p), emb_table.dtype),
        grid_spec=grid_spec,
        compiler_params=pltpu.CompilerParams(
            dimension_semantics=("parallel", "arbitrary"),
            vmem_limit_bytes=_VMEM_LIMIT_BYTES,
        ),
    )(ids, emb3)

    return out[:n, :emb].reshape(seq_len, batch, emb)
